# baseline (speedup 1.0000x reference)
import functools

import jax
import jax.numpy as jnp
from jax import lax
from jax.experimental import pallas as pl
from jax.experimental.pallas import tpu as pltpu
from jax.experimental.pallas import tpu_sc as plsc

B, C, T = 128, 2, 122880
T_OUT = 23552
NC, NS = 2, 16
NW = NC * NS
ROWS = B * C
RPW = ROWS // NW

SEG1 = (0, 40960, 2, 8, 0, 5120)
SEG2 = (81920, 36864, 1, 4, 10240, 9216)
SEG3_IN, SEG3_OUT, SEG3_LEN = 118784, 19456, 4096

IN_BUF = 40960

_mesh = plsc.VectorSubcoreMesh(core_axis_name="c", subcore_axis_name="s")


@functools.partial(
    pl.kernel,
    out_type=jax.ShapeDtypeStruct((B, C, T_OUT), jnp.float32),
    mesh=_mesh,
    scratch_types=[
        pltpu.VMEM((IN_BUF,), jnp.float32),
        pltpu.VMEM((T_OUT,), jnp.float32),
    ],
    compiler_params=pltpu.CompilerParams(
        needs_layout_passes=False,
        use_tc_tiling_on_sc=False,
    ),
)
def _decimate(strain_hbm, out_hbm, in_v, row_v):
    widx = lax.axis_index("s") * NC + lax.axis_index("c")
    iota = lax.iota(jnp.int32, 16)

    def do_row(r, carry):
        b = r // C
        ch = r % C

        pltpu.sync_copy(
            strain_hbm.at[b, ch, pl.ds(SEG3_IN, SEG3_LEN)],
            row_v.at[pl.ds(SEG3_OUT, SEG3_LEN)],
        )

        for in0, cin, nch, stride, o0, cout in (SEG1, SEG2):
            istride = iota * stride

            def do_chunk(k, c2, in0=in0, cin=cin, stride=stride, o0=o0,
                         cout=cout, istride=istride):
                pltpu.sync_copy(
                    strain_hbm.at[b, ch, pl.ds(in0 + k * cin, cin)],
                    in_v.at[pl.ds(0, cin)],
                )
                obase = o0 + k * cout

                def dec(i, c3, stride=stride, obase=obase, istride=istride):
                    for u in range(4):
                        idx = istride + (i * (64 * stride) + u * (16 * stride))
                        g = plsc.load_gather(in_v, [idx])
                        row_v[pl.ds(obase + i * 64 + u * 16, 16)] = g
                    return c3

                return lax.fori_loop(0, cout // 64, dec, c2)

            carry = lax.fori_loop(0, nch, do_chunk, carry)

        pltpu.sync_copy(row_v, out_hbm.at[b, ch])
        return carry

    lax.fori_loop(0, RPW, lambda i, c: do_row(widx * RPW + i, c), 0)


def kernel(strain):
    return _decimate(strain)


# speedup vs baseline: 1.1256x; 1.1256x over previous
import functools

import jax
import jax.numpy as jnp
from jax import lax
from jax.experimental import pallas as pl
from jax.experimental.pallas import tpu as pltpu
from jax.experimental.pallas import tpu_sc as plsc

B, C, T = 128, 2, 122880
T_OUT = 23552
NC, NS = 2, 16
NW = NC * NS
ROWS = B * C
RPW = ROWS // NW

IN_BUF = 20480

JOBS = tuple(
    [(k * 20480, 20480, 8, k * 2560, 2560) for k in range(4)]
    + [(81920 + k * 18432, 18432, 4, 10240 + k * 4608, 4608) for k in range(2)]
)
NJ = len(JOBS)
SEG3_IN, SEG3_OUT, SEG3_LEN = 118784, 19456, 4096

_mesh = plsc.VectorSubcoreMesh(core_axis_name="c", subcore_axis_name="s")


@functools.partial(
    pl.kernel,
    out_type=jax.ShapeDtypeStruct((B, C, T_OUT), jnp.float32),
    mesh=_mesh,
    scratch_types=[
        pltpu.VMEM((2, IN_BUF), jnp.float32),
        pltpu.VMEM((2, T_OUT), jnp.float32),
        pltpu.SemaphoreType.DMA,
        pltpu.SemaphoreType.DMA,
        pltpu.SemaphoreType.DMA,
        pltpu.SemaphoreType.DMA,
        pltpu.SemaphoreType.DMA,
        pltpu.SemaphoreType.DMA,
    ],
    compiler_params=pltpu.CompilerParams(
        needs_layout_passes=False,
        use_tc_tiling_on_sc=False,
    ),
)
def _decimate(strain_hbm, out_hbm, in_v, row_v, si0, si1, so0, so1, s30, s31):
    widx = lax.axis_index("s") * NC + lax.axis_index("c")
    iota = lax.iota(jnp.int32, 16)
    sin = (si0, si1)
    sout = (so0, so1)
    s3 = (s30, s31)

    def start_in(r, j, buf):
        in0, cin, _, _, _ = JOBS[j]
        b = r // C
        ch = r % C
        return pltpu.async_copy(
            strain_hbm.at[b, ch, pl.ds(in0, cin)],
            in_v.at[buf, pl.ds(0, cin)],
            sin[buf],
        )

    def dec_chunk(j, buf, p):
        _, _, stride, o0, cout = JOBS[j]
        istride = iota * stride

        def dec(i, c3):
            for u in range(4):
                idx = istride + (i * (64 * stride) + u * (16 * stride))
                g = plsc.load_gather(in_v.at[buf], [idx])
                row_v[p, pl.ds(o0 + i * 64 + u * 16, 16)] = g
            return c3

        lax.fori_loop(0, cout // 64, dec, 0)

    out_cps = [None, None]
    for r_local in range(RPW):
        r = widx * RPW + r_local
        b = r // C
        ch = r % C
        p = r_local & 1

        if r_local >= 2:
            out_cps[p].wait()

        c3 = pltpu.async_copy(
            strain_hbm.at[b, ch, pl.ds(SEG3_IN, SEG3_LEN)],
            row_v.at[p, pl.ds(SEG3_OUT, SEG3_LEN)],
            s3[p],
        )

        cps = [None] * NJ
        cps[0] = start_in(r, 0, 0)
        cps[1] = start_in(r, 1, 1)
        for j in range(NJ):
            buf = j & 1
            cps[j].wait()
            dec_chunk(j, buf, p)
            if j + 2 < NJ:
                cps[j + 2] = start_in(r, j + 2, buf)

        c3.wait()
        out_cps[p] = pltpu.async_copy(row_v.at[p], out_hbm.at[b, ch], sout[p])

    out_cps[0].wait()
    out_cps[1].wait()


def kernel(strain):
    return _decimate(strain)


# speedup vs baseline: 2.6956x; 2.3949x over previous
import functools

import jax
import jax.numpy as jnp
from jax import lax
from jax.experimental import pallas as pl
from jax.experimental.pallas import tpu as pltpu
from jax.experimental.pallas import tpu_sc as plsc

B, C, T = 128, 2, 122880
T_OUT = 23552
TH, LH = 960, 128
OH = 184
NC, NS = 2, 16
NW = NC * NS
BPW = B // NW

JOBS = tuple(
    [(128 * k, 128, 8, 2048 * k) for k in range(5)]
    + [(640 + 96 * k, 96, 4, 10240 + 3072 * k) for k in range(3)]
)
NJ = len(JOBS)
S3_TH, S3_CNT, S3_OH = 928, 32, 152

IN_TH = 128

_mesh = plsc.VectorSubcoreMesh(core_axis_name="c", subcore_axis_name="s")


@functools.partial(
    pl.kernel,
    out_type=jax.ShapeDtypeStruct((B, OH, C, LH), jnp.float32),
    mesh=_mesh,
    scratch_types=[
        pltpu.VMEM((2, IN_TH, C, LH), jnp.float32),
        pltpu.VMEM((OH, C, LH), jnp.float32),
        pltpu.SemaphoreType.DMA,
        pltpu.SemaphoreType.DMA,
        pltpu.SemaphoreType.DMA,
        pltpu.SemaphoreType.DMA,
    ],
    compiler_params=pltpu.CompilerParams(
        needs_layout_passes=False,
        use_tc_tiling_on_sc=False,
    ),
)
def _decimate(strain_hbm, out_hbm, in_v, row_v, si0, si1, so, s3):
    widx = lax.axis_index("s") * NC + lax.axis_index("c")
    iota = lax.iota(jnp.int32, 16)
    tl8 = iota * 8
    tl4a = iota * 4
    tl4b = iota * 4 + 64
    sin = (si0, si1)

    def start_in(b, j, buf):
        th0, cnt, _, _ = JOBS[j]
        return pltpu.async_copy(
            strain_hbm.at[b, pl.ds(th0, cnt)],
            in_v.at[buf, pl.ds(0, cnt)],
            sin[buf],
        )

    def dec_chunk(j, buf):
        _, cnt, stride, o0 = JOBS[j]
        for c in range(C):
            cvec = iota * 0 + c
            if stride == 8:
                def dec8(i, cr, buf=buf, c=c, cvec=cvec, o0=o0):
                    for u in range(4):
                        th = i * 4 + u
                        g = plsc.load_gather(
                            in_v.at[buf], [iota * 0 + th, cvec, tl8]
                        )
                        o = o0 + th * 16
                        row_v[o >> 7, c, pl.ds(o & 127, 16)] = g
                    return cr

                lax.fori_loop(0, cnt // 4, dec8, 0)
            else:
                def dec4(i, cr, buf=buf, c=c, cvec=cvec, o0=o0):
                    for u in range(2):
                        th = i * 2 + u
                        thv = iota * 0 + th
                        o = o0 + th * 32
                        g = plsc.load_gather(in_v.at[buf], [thv, cvec, tl4a])
                        row_v[o >> 7, c, pl.ds(o & 127, 16)] = g
                        g = plsc.load_gather(in_v.at[buf], [thv, cvec, tl4b])
                        o = o + 16
                        row_v[o >> 7, c, pl.ds(o & 127, 16)] = g
                    return cr

                lax.fori_loop(0, cnt // 2, dec4, 0)

    out_cp = None
    for b_local in range(BPW):
        b = widx * BPW + b_local

        cps = [None] * NJ
        cps[0] = start_in(b, 0, 0)
        cps[1] = start_in(b, 1, 1)

        if out_cp is not None:
            out_cp.wait()

        c3 = pltpu.async_copy(
            strain_hbm.at[b, pl.ds(S3_TH, S3_CNT)],
            row_v.at[pl.ds(S3_OH, S3_CNT)],
            s3,
        )

        for j in range(NJ):
            buf = j & 1
            cps[j].wait()
            dec_chunk(j, buf)
            if j + 2 < NJ:
                cps[j + 2] = start_in(b, j + 2, buf)

        c3.wait()
        out_cp = pltpu.async_copy(row_v, out_hbm.at[b], so)

    out_cp.wait()


def kernel(strain):
    a = strain.reshape(B, C, TH, LH).transpose(0, 2, 1, 3)
    y = _decimate(a)
    return y.transpose(0, 2, 1, 3).reshape(B, C, T_OUT)
